# Initial kernel scaffold; baseline (speedup 1.0000x reference)
import jax
import jax.numpy as jnp
from jax.experimental import pallas as pl


def kernel(features, edge_index, params):
    raise NotImplementedError("write your pallas kernel here")


# speedup vs baseline: 5.0300x; 5.0300x over previous
import functools

import jax
import jax.numpy as jnp
from jax import lax
from jax.experimental import pallas as pl
from jax.experimental.pallas import tpu as pltpu
from jax.experimental.pallas import tpu_sc as plsc

N = 10000
E = 320000
H = 128
NC = 2
NS = 16
C = 128
CHUNKS = E // C
NP = 10240
ROWS_PER_SUBCORE = NP // NS
ZR = 128

_TC_R = 400
_TC_GRID = N // _TC_R


def _sc_mesh():
    return plsc.VectorSubcoreMesh(core_axis_name="c", subcore_axis_name="s")


def _zero_spmem(zero_v, table_sh, sid):
    base = sid * ROWS_PER_SUBCORE
    full, rem = divmod(ROWS_PER_SUBCORE, ZR)
    for k in range(full):
        pltpu.sync_copy(zero_v, table_sh.at[pl.ds(base + k * ZR, ZR)])
    if rem:
        pltpu.sync_copy(zero_v.at[pl.ds(0, rem)],
                        table_sh.at[pl.ds(base + full * ZR, rem)])


def _sc_degree(dst3, ones_src, zerosH):
    @functools.partial(
        pl.kernel,
        mesh=_sc_mesh(),
        out_type=jax.ShapeDtypeStruct((NC, NP, H), jnp.float32),
        scratch_types=[
            pltpu.VMEM((1, C), jnp.int32),
            pltpu.VMEM((C, H), jnp.float32),
            pltpu.VMEM((ZR, H), jnp.float32),
            pltpu.VMEM_SHARED((NP, H), jnp.float32),
        ],
    )
    def deg_kernel(dst_hbm, ones_hbm, z_hbm, out_hbm, idx_v, ones_v, zero_v,
                   deg_sh):
        cid = lax.axis_index("c")
        sid = lax.axis_index("s")
        pltpu.sync_copy(z_hbm, zero_v)
        pltpu.sync_copy(ones_hbm, ones_v)
        _zero_spmem(zero_v, deg_sh, sid)
        plsc.subcore_barrier()
        start = sid * NC + cid

        @pl.loop(start, CHUNKS, step=NC * NS)
        def _(c):
            pltpu.sync_copy(dst_hbm.at[c], idx_v)
            pltpu.sync_copy(ones_v, deg_sh.at[idx_v.at[0]], add=True)

        plsc.subcore_barrier()
        base = sid * ROWS_PER_SUBCORE
        pltpu.sync_copy(deg_sh.at[pl.ds(base, ROWS_PER_SUBCORE)],
                        out_hbm.at[cid, pl.ds(base, ROWS_PER_SUBCORE)])

    return deg_kernel(dst3, ones_src, zerosH)


def _sc_segment_sum(x, src3, dst3, zerosH):
    @functools.partial(
        pl.kernel,
        mesh=_sc_mesh(),
        out_type=jax.ShapeDtypeStruct((NC, NP, H), jnp.float32),
        scratch_types=[
            pltpu.VMEM((1, C), jnp.int32),
            pltpu.VMEM((1, C), jnp.int32),
            pltpu.VMEM((C, H), jnp.float32),
            pltpu.VMEM((ZR, H), jnp.float32),
            pltpu.VMEM_SHARED((NP, H), jnp.float32),
            pltpu.SemaphoreType.DMA,
        ],
    )
    def agg_kernel(x_hbm, src_hbm, dst_hbm, z_hbm, out_hbm, src_v, dst_v,
                   rows_v, zero_v, agg_sh, sem):
        cid = lax.axis_index("c")
        sid = lax.axis_index("s")
        pltpu.sync_copy(z_hbm, zero_v)
        _zero_spmem(zero_v, agg_sh, sid)
        plsc.subcore_barrier()
        start = sid * NC + cid

        @pl.loop(start, CHUNKS, step=NC * NS)
        def _(c):
            pltpu.sync_copy(src_hbm.at[c], src_v)
            pltpu.sync_copy(dst_hbm.at[c], dst_v)
            pltpu.async_copy(x_hbm.at[src_v.at[0]], rows_v, sem).wait()
            pltpu.sync_copy(rows_v, agg_sh.at[dst_v.at[0]], add=True)

        plsc.subcore_barrier()
        base = sid * ROWS_PER_SUBCORE
        pltpu.sync_copy(agg_sh.at[pl.ds(base, ROWS_PER_SUBCORE)],
                        out_hbm.at[cid, pl.ds(base, ROWS_PER_SUBCORE)])

    return agg_kernel(x, src3, dst3, zerosH)


def _tc_ffn_ln(x, w1, b1, w2, b2, g, b):
    def body(x_ref, w1_ref, b1_ref, w2_ref, b2_ref, g_ref, b_ref, o_ref):
        xv = x_ref[...]
        h = jnp.maximum(
            jnp.dot(xv, w1_ref[...], preferred_element_type=jnp.float32)
            + b1_ref[...], 0.0)
        ff = jnp.dot(h, w2_ref[...], preferred_element_type=jnp.float32) \
            + b2_ref[...]
        y = ff + xv
        mu = jnp.mean(y, axis=-1, keepdims=True)
        var = jnp.mean((y - mu) ** 2, axis=-1, keepdims=True)
        o_ref[...] = (y - mu) * lax.rsqrt(var + 1e-5) * g_ref[...] + b_ref[...]

    full = lambda i: (0, 0)
    return pl.pallas_call(
        body,
        grid=(_TC_GRID,),
        in_specs=[
            pl.BlockSpec((_TC_R, H), lambda i: (i, 0)),
            pl.BlockSpec((H, H), full),
            pl.BlockSpec((1, H), full),
            pl.BlockSpec((H, H), full),
            pl.BlockSpec((1, H), full),
            pl.BlockSpec((1, H), full),
            pl.BlockSpec((1, H), full),
        ],
        out_specs=pl.BlockSpec((_TC_R, H), lambda i: (i, 0)),
        out_shape=jax.ShapeDtypeStruct((N, H), jnp.float32),
    )(x, w1, b1, w2, b2, g, b)


def _tc_gcn_ln(partial, degp, x, w, bias, g, b):
    def body(p_ref, d_ref, x_ref, w_ref, bias_ref, g_ref, b_ref, o_ref):
        agg = p_ref[0] + p_ref[1]
        deg = d_ref[0, :, 0:1] + d_ref[1, :, 0:1]
        agg = agg / jnp.maximum(deg, 1.0)
        gcn = jnp.maximum(
            jnp.dot(agg, w_ref[...], preferred_element_type=jnp.float32)
            + bias_ref[...], 0.0)
        y = gcn + x_ref[...]
        mu = jnp.mean(y, axis=-1, keepdims=True)
        var = jnp.mean((y - mu) ** 2, axis=-1, keepdims=True)
        o_ref[...] = (y - mu) * lax.rsqrt(var + 1e-5) * g_ref[...] + b_ref[...]

    full = lambda i: (0, 0)
    return pl.pallas_call(
        body,
        grid=(_TC_GRID,),
        in_specs=[
            pl.BlockSpec((NC, _TC_R, H), lambda i: (0, i, 0)),
            pl.BlockSpec((NC, _TC_R, H), lambda i: (0, i, 0)),
            pl.BlockSpec((_TC_R, H), lambda i: (i, 0)),
            pl.BlockSpec((H, H), full),
            pl.BlockSpec((1, H), full),
            pl.BlockSpec((1, H), full),
            pl.BlockSpec((1, H), full),
        ],
        out_specs=pl.BlockSpec((_TC_R, H), lambda i: (i, 0)),
        out_shape=jax.ShapeDtypeStruct((N, H), jnp.float32),
    )(partial, degp, x, w, bias, g, b)


def kernel(features, edge_index, params):
    src3 = edge_index[0].astype(jnp.int32).reshape(CHUNKS, 1, C)
    dst3 = edge_index[1].astype(jnp.int32).reshape(CHUNKS, 1, C)
    onesH = jnp.ones((C, H), jnp.float32)
    zerosH = jnp.zeros((ZR, H), jnp.float32)

    degp = _sc_degree(dst3, onesH, zerosH)

    out = features
    for i in range(2):
        p = params[f"l{i}"]
        out = _tc_ffn_ln(out, p["w1"], p["b1"].reshape(1, H),
                         p["w2"], p["b2"].reshape(1, H),
                         p["ln1_g"].reshape(1, H), p["ln1_b"].reshape(1, H))
        partial = _sc_segment_sum(out, src3, dst3, zerosH)
        out = _tc_gcn_ln(partial, degp, out, p["gcn_w"],
                         p["gcn_b"].reshape(1, H),
                         p["ln2_g"].reshape(1, H), p["ln2_b"].reshape(1, H))
    return out
